# baseline (speedup 1.0000x reference)
import jax
import jax.numpy as jnp
from jax.experimental import pallas as pl
from jax.experimental.pallas import tpu as pltpu

_B, _K, _H, _W = 16, 17, 80, 80
_R = (_H * _W) // 128


def _body(s2k_ref, xi_ref, xr_ref, xs_ref, xc_ref, ti_ref, tr_ref, tc_ref,
          out_ref):
    b = pl.program_id(0)

    ti = ti_ref[0]
    tsum = jnp.sum(ti, axis=0)
    bce_mask = tsum > 0.5
    bt = ti[:_K]

    x = xi_ref[0]
    per = jnp.maximum(x, 0.0) - x * bt + jnp.log1p(jnp.exp(-jnp.abs(x)))
    ce_part = jnp.sum(jnp.where(bce_mask[None], per, 0.0))
    nsel_part = float(_K) * jnp.sum(bce_mask.astype(jnp.float32))

    reg_mask = bt > 0.5
    xr = xr_ref[0]
    tr = tr_ref[0]
    d = (xr[:, 0] - tr[:, 0]) ** 2 + (xr[:, 1] - tr[:, 1]) ** 2
    norm = jnp.sqrt(jnp.where(reg_mask, d, 1.0))
    lap = 0.694 + xs_ref[0] + norm * jnp.exp(-xs_ref[0])
    reg_part = jnp.sum(jnp.where(reg_mask, lap, 0.0))

    sc = jnp.abs(xc_ref[0] - tc_ref[0] * s2k_ref[...])
    sc_part = jnp.sum(jnp.where(reg_mask, sc, 0.0))

    @pl.when(b == 0)
    def _():
        out_ref[0] = ce_part
        out_ref[1] = nsel_part
        out_ref[2] = reg_part
        out_ref[3] = sc_part

    @pl.when(b != 0)
    def _():
        out_ref[0] += ce_part
        out_ref[1] += nsel_part
        out_ref[2] += reg_part
        out_ref[3] += sc_part


def kernel(x_intensity, x_reg, x_spread, x_scale, t_intensity, t_reg,
           t_scale, scales_to_kp):
    xi = x_intensity.reshape(_B, _K, _R, 128)
    xr = x_reg.reshape(_B, _K, 2, _R, 128)
    xs = x_spread.reshape(_B, _K, _R, 128)
    xc = x_scale.reshape(_B, _K, _R, 128)
    ti = t_intensity.reshape(_B, _K + 1, _R, 128)
    tr = t_reg.reshape(_B, _K, 2, _R, 128)
    tc = t_scale.reshape(_B, _K, _R, 128)
    s2k = jnp.broadcast_to(scales_to_kp.reshape(_K, 1, 1), (_K, 1, 128))

    sums = pl.pallas_call(
        _body,
        grid=(_B,),
        in_specs=[
            pl.BlockSpec((_K, 1, 128), lambda b: (0, 0, 0)),
            pl.BlockSpec((1, _K, _R, 128), lambda b: (b, 0, 0, 0)),
            pl.BlockSpec((1, _K, 2, _R, 128), lambda b: (b, 0, 0, 0, 0)),
            pl.BlockSpec((1, _K, _R, 128), lambda b: (b, 0, 0, 0)),
            pl.BlockSpec((1, _K, _R, 128), lambda b: (b, 0, 0, 0)),
            pl.BlockSpec((1, _K + 1, _R, 128), lambda b: (b, 0, 0, 0)),
            pl.BlockSpec((1, _K, 2, _R, 128), lambda b: (b, 0, 0, 0, 0)),
            pl.BlockSpec((1, _K, _R, 128), lambda b: (b, 0, 0, 0)),
        ],
        out_specs=pl.BlockSpec(memory_space=pltpu.SMEM),
        out_shape=jax.ShapeDtypeStruct((4,), jnp.float32),
    )(s2k, xi, xr, xs, xc, ti, tr, tc)

    ce_loss = sums[0] / sums[1]
    reg_loss = sums[2] / 1000.0 / _B
    scale_loss = sums[3] / 1000.0 / _B
    return (ce_loss, reg_loss, scale_loss)
